# baseline (speedup 1.0000x reference)
import functools

import jax
import jax.numpy as jnp
from jax import lax
from jax.experimental import pallas as pl
from jax.experimental.pallas import tpu as pltpu
from jax.experimental.pallas import tpu_sc as plsc

M = 2048
H = 1024
I = 768
E = 8
K = 2
P = M * K
CH = 256
NCH = M // CH
BM = 256
CWL = 128

NC, NS = 2, 16
NW = NC * NS
TPW = M // NW


def _router_body(x_ref, gate_ref, d1_ref, d2_ref, cw1_ref, cw2_ref, xb_ref):
    x = x_ref[...]
    gate_w = gate_ref[...]
    logits = lax.dot_general(x, gate_w, (((1,), (1,)), ((), ())),
                             preferred_element_type=jnp.float32)
    idxE = lax.broadcasted_iota(jnp.int32, (M, E), 1)
    m1 = jnp.max(logits, axis=1, keepdims=True)
    id1 = jnp.min(jnp.where(logits == m1, idxE, E), axis=1, keepdims=True)
    masked = jnp.where(idxE == id1, -jnp.inf, logits)
    m2 = jnp.max(masked, axis=1, keepdims=True)
    id2 = jnp.min(jnp.where(masked == m2, idxE, E), axis=1, keepdims=True)
    r = jnp.exp(m2 - m1)
    t1 = 1.0 / (1.0 + r)
    t2 = r / (1.0 + r)

    nh1 = (idxE == id1).astype(jnp.float32)
    nh2 = (idxE == id2).astype(jnp.float32)
    a = nh1 + nh2

    inc = a
    sh = 1
    while sh < M:
        inc = inc + jnp.concatenate(
            [jnp.zeros((sh, E), jnp.float32), inc[:M - sh]], axis=0)
        sh *= 2
    c_excl = inc - a
    counts = inc[M - 1:M, :]
    off_inc = counts
    sh = 1
    while sh < E:
        off_inc = off_inc + jnp.concatenate(
            [jnp.zeros((1, sh), jnp.float32), off_inc[:, :E - sh]], axis=1)
        sh *= 2
    offsets = off_inc - counts

    dmat = c_excl + offsets
    d1_ref[...] = jnp.sum(nh1 * dmat, axis=1).astype(jnp.int32)
    d2_ref[...] = jnp.sum(nh2 * dmat, axis=1).astype(jnp.int32)

    zpad = jnp.zeros((M, CWL - E), dtype=jnp.float32)
    cw1_ref[...] = jnp.concatenate([nh1 * t1, zpad], axis=1)
    cw2_ref[...] = jnp.concatenate([nh2 * t2, zpad], axis=1)
    xb_ref[...] = x.astype(jnp.bfloat16)


def _router_call(x, gate_w):
    return pl.pallas_call(
        _router_body,
        grid=(1,),
        in_specs=[
            pl.BlockSpec((M, H), lambda i: (0, 0)),
            pl.BlockSpec((E, H), lambda i: (0, 0)),
        ],
        out_specs=[
            pl.BlockSpec((M,), lambda i: (0,)),
            pl.BlockSpec((M,), lambda i: (0,)),
            pl.BlockSpec((M, CWL), lambda i: (0, 0)),
            pl.BlockSpec((M, CWL), lambda i: (0, 0)),
            pl.BlockSpec((M, H), lambda i: (0, 0)),
        ],
        out_shape=[
            jax.ShapeDtypeStruct((M,), jnp.int32),
            jax.ShapeDtypeStruct((M,), jnp.int32),
            jax.ShapeDtypeStruct((M, CWL), jnp.float32),
            jax.ShapeDtypeStruct((M, CWL), jnp.float32),
            jax.ShapeDtypeStruct((M, H), jnp.bfloat16),
        ],
        compiler_params=pltpu.CompilerParams(
            vmem_limit_bytes=100 * 1024 * 1024,
        ),
    )(x, gate_w)


@functools.lru_cache(maxsize=None)
def _sc_mesh():
    return plsc.VectorSubcoreMesh(core_axis_name="c", subcore_axis_name="s",
                                  num_cores=NC, num_subcores=NS)


@functools.lru_cache(maxsize=None)
def _dispatch_fn():
    @functools.partial(
        pl.kernel,
        out_type=[
            jax.ShapeDtypeStruct((P, H // 2), jnp.int32),
            jax.ShapeDtypeStruct((P, CWL), jnp.float32),
        ],
        mesh=_sc_mesh(),
        scratch_types=[
            pltpu.VMEM((TPW, H // 2), jnp.int32),
            pltpu.VMEM((TPW, CWL), jnp.float32),
            pltpu.VMEM((TPW,), jnp.int32),
            pltpu.VMEM((TPW,), jnp.int32),
            pltpu.SemaphoreType.DMA,
            pltpu.SemaphoreType.DMA,
        ],
    )
    def _dispatch(xb_hbm, d1_hbm, d2_hbm, cw1_hbm, cw2_hbm, xs_hbm, cws_hbm,
                  xrows, cwbuf, idx1, idx2, sem1, sem2):
        wid = lax.axis_index("s") * NC + lax.axis_index("c")
        base = wid * TPW
        pltpu.sync_copy(d1_hbm.at[pl.ds(base, TPW)], idx1)
        pltpu.sync_copy(d2_hbm.at[pl.ds(base, TPW)], idx2)
        pltpu.sync_copy(xb_hbm.at[pl.ds(base, TPW)], xrows)
        c1 = pltpu.async_copy(xrows, xs_hbm.at[idx1], sem1)
        c2 = pltpu.async_copy(xrows, xs_hbm.at[idx2], sem2)
        pltpu.sync_copy(cw1_hbm.at[pl.ds(base, TPW)], cwbuf)
        c1.wait()
        pltpu.async_copy(cwbuf, cws_hbm.at[idx1], sem1).wait()
        pltpu.sync_copy(cw2_hbm.at[pl.ds(base, TPW)], cwbuf)
        c2.wait()
        pltpu.async_copy(cwbuf, cws_hbm.at[idx2], sem2).wait()

    return _dispatch


def _gmm_body(xs_ref, cw_ref, w13_ref, w2_ref, ys_ref):
    xb = xs_ref[...]
    cw = cw_ref[...]
    ys_ref[...] = jnp.zeros((BM, H), jnp.float32)
    for e in range(E):
        sc = cw[:, e]

        @pl.when(jnp.any(sc != 0.0))
        def _(e=e, sc=sc):
            gu = lax.dot_general(xb, w13_ref[e], (((1,), (1,)), ((), ())),
                                 preferred_element_type=jnp.float32)
            g = gu[:, :I]
            u = gu[:, I:]
            h = (g * (1.0 / (1.0 + jnp.exp(-g)))) * u * sc[:, None]
            ys_ref[...] += lax.dot_general(
                h.astype(jnp.bfloat16), w2_ref[e], (((1,), (1,)), ((), ())),
                preferred_element_type=jnp.float32)


def _gmm_call(xs, cws, w13b, w2b):
    return pl.pallas_call(
        _gmm_body,
        grid=(P // BM,),
        in_specs=[
            pl.BlockSpec((BM, H), lambda i: (i, 0)),
            pl.BlockSpec((BM, CWL), lambda i: (i, 0)),
            pl.BlockSpec((E, 2 * I, H), lambda i: (0, 0, 0)),
            pl.BlockSpec((E, H, I), lambda i: (0, 0, 0)),
        ],
        out_specs=pl.BlockSpec((BM, H), lambda i: (i, 0)),
        out_shape=jax.ShapeDtypeStruct((P, H), jnp.float32),
        compiler_params=pltpu.CompilerParams(
            vmem_limit_bytes=100 * 1024 * 1024,
        ),
    )(xs, cws, w13b, w2b)


TPW2 = TPW // 2


@functools.lru_cache(maxsize=None)
def _combine_fn():
    @functools.partial(
        pl.kernel,
        out_type=jax.ShapeDtypeStruct((M, H), jnp.float32),
        mesh=_sc_mesh(),
        scratch_types=[
            pltpu.VMEM((TPW2, H), jnp.float32),
            pltpu.VMEM((TPW2, H), jnp.float32),
            pltpu.VMEM((TPW,), jnp.int32),
            pltpu.VMEM((TPW,), jnp.int32),
            pltpu.SemaphoreType.DMA,
            pltpu.SemaphoreType.DMA,
        ],
    )
    def _combine(ys_hbm, d1_hbm, d2_hbm, out_hbm, acc, buf2, idx1, idx2,
                 sem1, sem2):
        wid = lax.axis_index("s") * NC + lax.axis_index("c")
        base = wid * TPW
        pltpu.sync_copy(d1_hbm.at[pl.ds(base, TPW)], idx1)
        pltpu.sync_copy(d2_hbm.at[pl.ds(base, TPW)], idx2)
        for half in range(2):
            ia = idx1.at[pl.ds(half * TPW2, TPW2)]
            ib = idx2.at[pl.ds(half * TPW2, TPW2)]
            c1 = pltpu.async_copy(ys_hbm.at[ia], acc, sem1)
            c2 = pltpu.async_copy(ys_hbm.at[ib], buf2, sem2)
            c1.wait()
            c2.wait()

            def body(c, _):
                col = c * 16
                for r in range(TPW2):
                    acc[r, pl.ds(col, 16)] = (acc[r, pl.ds(col, 16)]
                                              + buf2[r, pl.ds(col, 16)])
                return 0

            lax.fori_loop(0, H // 16, body, 0)
            pltpu.sync_copy(acc,
                            out_hbm.at[pl.ds(base + half * TPW2, TPW2)])

    return _combine


@jax.jit
def kernel(hidden_states, gate_w, w13, w2):
    x = hidden_states.reshape(-1, H)
    w13b = w13.astype(jnp.bfloat16)
    w2b = w2.astype(jnp.bfloat16)
    d1, d2, cw1, cw2, xb = _router_call(x, gate_w)
    xb_i32 = lax.bitcast_convert_type(xb.reshape(M, H // 2, 2), jnp.int32)
    xs_i32, cws = _dispatch_fn()(xb_i32, d1, d2, cw1, cw2)
    xs = lax.bitcast_convert_type(xs_i32, jnp.bfloat16).reshape(P, H)
    ys = _gmm_call(xs, cws, w13b, w2b)
    out = _combine_fn()(ys, d1, d2)
    return out.reshape(hidden_states.shape)


# speedup vs baseline: 1.7053x; 1.7053x over previous
import functools

import jax
import jax.numpy as jnp
from jax import lax
from jax.experimental import pallas as pl
from jax.experimental.pallas import tpu as pltpu
from jax.experimental.pallas import tpu_sc as plsc

M = 2048
H = 1024
I = 768
E = 8
K = 2
P = M * K
CH = 256
NCH = M // CH
BM = 256
CWL = 128

NC, NS = 2, 16
NW = NC * NS
TPW = M // NW


def _router_body(x_ref, gate_ref, d1_ref, d2_ref, cw1_ref, cw2_ref):
    x = x_ref[...]
    gate_w = gate_ref[...]
    logits = lax.dot_general(x, gate_w, (((1,), (1,)), ((), ())),
                             preferred_element_type=jnp.float32)
    idxE = lax.broadcasted_iota(jnp.int32, (M, E), 1)
    m1 = jnp.max(logits, axis=1, keepdims=True)
    id1 = jnp.min(jnp.where(logits == m1, idxE, E), axis=1, keepdims=True)
    masked = jnp.where(idxE == id1, -jnp.inf, logits)
    m2 = jnp.max(masked, axis=1, keepdims=True)
    id2 = jnp.min(jnp.where(masked == m2, idxE, E), axis=1, keepdims=True)
    r = jnp.exp(m2 - m1)
    t1 = 1.0 / (1.0 + r)
    t2 = r / (1.0 + r)

    nh1 = (idxE == id1).astype(jnp.float32)
    nh2 = (idxE == id2).astype(jnp.float32)
    a = nh1 + nh2

    inc = a
    sh = 1
    while sh < M:
        inc = inc + jnp.concatenate(
            [jnp.zeros((sh, E), jnp.float32), inc[:M - sh]], axis=0)
        sh *= 2
    c_excl = inc - a
    counts = inc[M - 1:M, :]
    off_inc = counts
    sh = 1
    while sh < E:
        off_inc = off_inc + jnp.concatenate(
            [jnp.zeros((1, sh), jnp.float32), off_inc[:, :E - sh]], axis=1)
        sh *= 2
    offsets = off_inc - counts

    dmat = c_excl + offsets
    d1_ref[...] = jnp.sum(nh1 * dmat, axis=1).astype(jnp.int32)
    d2_ref[...] = jnp.sum(nh2 * dmat, axis=1).astype(jnp.int32)

    zpad = jnp.zeros((M, CWL - E), dtype=jnp.float32)
    cw1_ref[...] = jnp.concatenate([nh1 * t1, zpad], axis=1)
    cw2_ref[...] = jnp.concatenate([nh2 * t2, zpad], axis=1)


def _router_call(x, gate_w):
    return pl.pallas_call(
        _router_body,
        grid=(1,),
        in_specs=[
            pl.BlockSpec((M, H), lambda i: (0, 0)),
            pl.BlockSpec((E, H), lambda i: (0, 0)),
        ],
        out_specs=[
            pl.BlockSpec((M,), lambda i: (0,)),
            pl.BlockSpec((M,), lambda i: (0,)),
            pl.BlockSpec((M, CWL), lambda i: (0, 0)),
            pl.BlockSpec((M, CWL), lambda i: (0, 0)),
        ],
        out_shape=[
            jax.ShapeDtypeStruct((M,), jnp.int32),
            jax.ShapeDtypeStruct((M,), jnp.int32),
            jax.ShapeDtypeStruct((M, CWL), jnp.float32),
            jax.ShapeDtypeStruct((M, CWL), jnp.float32),
        ],
        compiler_params=pltpu.CompilerParams(
            vmem_limit_bytes=100 * 1024 * 1024,
        ),
    )(x, gate_w)


@functools.lru_cache(maxsize=None)
def _sc_mesh():
    return plsc.VectorSubcoreMesh(core_axis_name="c", subcore_axis_name="s",
                                  num_cores=NC, num_subcores=NS)


@functools.lru_cache(maxsize=None)
def _dispatch_fn():
    @functools.partial(
        pl.kernel,
        out_type=[
            jax.ShapeDtypeStruct((P, H), jnp.float32),
            jax.ShapeDtypeStruct((P, CWL), jnp.float32),
        ],
        mesh=_sc_mesh(),
        scratch_types=[
            pltpu.VMEM((TPW, H), jnp.float32),
            pltpu.VMEM((TPW, CWL), jnp.float32),
            pltpu.VMEM((TPW,), jnp.int32),
            pltpu.VMEM((TPW,), jnp.int32),
            pltpu.SemaphoreType.DMA,
            pltpu.SemaphoreType.DMA,
        ],
    )
    def _dispatch(x_hbm, d1_hbm, d2_hbm, cw1_hbm, cw2_hbm, xs_hbm, cws_hbm,
                  xrows, cwbuf, idx1, idx2, sem1, sem2):
        wid = lax.axis_index("s") * NC + lax.axis_index("c")
        base = wid * TPW
        pltpu.sync_copy(d1_hbm.at[pl.ds(base, TPW)], idx1)
        pltpu.sync_copy(d2_hbm.at[pl.ds(base, TPW)], idx2)
        pltpu.sync_copy(x_hbm.at[pl.ds(base, TPW)], xrows)
        c1 = pltpu.async_copy(xrows, xs_hbm.at[idx1], sem1)
        c2 = pltpu.async_copy(xrows, xs_hbm.at[idx2], sem2)
        pltpu.sync_copy(cw1_hbm.at[pl.ds(base, TPW)], cwbuf)
        c1.wait()
        pltpu.async_copy(cwbuf, cws_hbm.at[idx1], sem1).wait()
        pltpu.sync_copy(cw2_hbm.at[pl.ds(base, TPW)], cwbuf)
        c2.wait()
        pltpu.async_copy(cwbuf, cws_hbm.at[idx2], sem2).wait()

    return _dispatch


def _gmm_body(xs_ref, cw_ref, w13_ref, w2_ref, ys_ref):
    xb = xs_ref[...].astype(jnp.bfloat16)
    cw = cw_ref[...]
    ys_ref[...] = jnp.zeros((BM, H), jnp.float32)
    for e in range(E):
        sc = cw[:, e]

        @pl.when(jnp.any(sc != 0.0))
        def _(e=e, sc=sc):
            gu = lax.dot_general(xb, w13_ref[e], (((1,), (1,)), ((), ())),
                                 preferred_element_type=jnp.float32)
            g = gu[:, :I]
            u = gu[:, I:]
            h = (g * (1.0 / (1.0 + jnp.exp(-g)))) * u * sc[:, None]
            ys_ref[...] += lax.dot_general(
                h.astype(jnp.bfloat16), w2_ref[e], (((1,), (1,)), ((), ())),
                preferred_element_type=jnp.float32)


def _gmm_call(xs, cws, w13b, w2b):
    return pl.pallas_call(
        _gmm_body,
        grid=(P // BM,),
        in_specs=[
            pl.BlockSpec((BM, H), lambda i: (i, 0)),
            pl.BlockSpec((BM, CWL), lambda i: (i, 0)),
            pl.BlockSpec((E, 2 * I, H), lambda i: (0, 0, 0)),
            pl.BlockSpec((E, H, I), lambda i: (0, 0, 0)),
        ],
        out_specs=pl.BlockSpec((BM, H), lambda i: (i, 0)),
        out_shape=jax.ShapeDtypeStruct((P, H), jnp.float32),
        compiler_params=pltpu.CompilerParams(
            vmem_limit_bytes=100 * 1024 * 1024,
        ),
    )(xs, cws, w13b, w2b)


TPW2 = TPW // 2


@functools.lru_cache(maxsize=None)
def _combine_fn():
    @functools.partial(
        pl.kernel,
        out_type=jax.ShapeDtypeStruct((M, H), jnp.float32),
        mesh=_sc_mesh(),
        scratch_types=[
            pltpu.VMEM((TPW2, H), jnp.float32),
            pltpu.VMEM((TPW2, H), jnp.float32),
            pltpu.VMEM((TPW,), jnp.int32),
            pltpu.VMEM((TPW,), jnp.int32),
            pltpu.SemaphoreType.DMA,
            pltpu.SemaphoreType.DMA,
        ],
    )
    def _combine(ys_hbm, d1_hbm, d2_hbm, out_hbm, acc, buf2, idx1, idx2,
                 sem1, sem2):
        wid = lax.axis_index("s") * NC + lax.axis_index("c")
        base = wid * TPW
        pltpu.sync_copy(d1_hbm.at[pl.ds(base, TPW)], idx1)
        pltpu.sync_copy(d2_hbm.at[pl.ds(base, TPW)], idx2)
        for half in range(2):
            ia = idx1.at[pl.ds(half * TPW2, TPW2)]
            ib = idx2.at[pl.ds(half * TPW2, TPW2)]
            c1 = pltpu.async_copy(ys_hbm.at[ia], acc, sem1)
            c2 = pltpu.async_copy(ys_hbm.at[ib], buf2, sem2)
            c1.wait()
            c2.wait()

            def body(c, _):
                col = c * 16
                for r in range(TPW2):
                    acc[r, pl.ds(col, 16)] = (acc[r, pl.ds(col, 16)]
                                              + buf2[r, pl.ds(col, 16)])
                return 0

            lax.fori_loop(0, H // 16, body, 0)
            pltpu.sync_copy(acc,
                            out_hbm.at[pl.ds(base + half * TPW2, TPW2)])

    return _combine


@jax.jit
def kernel(hidden_states, gate_w, w13, w2):
    x = hidden_states.reshape(-1, H)
    w13b = w13.astype(jnp.bfloat16)
    w2b = w2.astype(jnp.bfloat16)
    d1, d2, cw1, cw2 = _router_call(x, gate_w)
    xs, cws = _dispatch_fn()(x, d1, d2, cw1, cw2)
    ys = _gmm_call(xs, cws, w13b, w2b)
    out = _combine_fn()(ys, d1, d2)
    return out.reshape(hidden_states.shape)


# speedup vs baseline: 2.0007x; 1.1732x over previous
import functools

import jax
import jax.numpy as jnp
from jax import lax
from jax.experimental import pallas as pl
from jax.experimental.pallas import tpu as pltpu
from jax.experimental.pallas import tpu_sc as plsc

M = 2048
H = 1024
I = 768
E = 8
K = 2
P = M * K
CH = 256
NCH = M // CH
BM = 256
CWL = 128
NB = P // BM
NT = NB + E - 1

NC, NS = 2, 16
NW = NC * NS
TPW = M // NW


def _router_body(x_ref, gate_ref, d1_ref, d2_ref, cw1_ref, cw2_ref,
                 tmeta_ref):
    x = x_ref[...]
    gate_w = gate_ref[...]
    logits = lax.dot_general(x, gate_w, (((1,), (1,)), ((), ())),
                             preferred_element_type=jnp.float32)
    idxE = lax.broadcasted_iota(jnp.int32, (M, E), 1)
    m1 = jnp.max(logits, axis=1, keepdims=True)
    id1 = jnp.min(jnp.where(logits == m1, idxE, E), axis=1, keepdims=True)
    masked = jnp.where(idxE == id1, -jnp.inf, logits)
    m2 = jnp.max(masked, axis=1, keepdims=True)
    id2 = jnp.min(jnp.where(masked == m2, idxE, E), axis=1, keepdims=True)
    r = jnp.exp(m2 - m1)
    t1 = 1.0 / (1.0 + r)
    t2 = r / (1.0 + r)

    nh1 = (idxE == id1).astype(jnp.float32)
    nh2 = (idxE == id2).astype(jnp.float32)
    a = nh1 + nh2

    inc = a
    sh = 1
    while sh < M:
        inc = inc + jnp.concatenate(
            [jnp.zeros((sh, E), jnp.float32), inc[:M - sh]], axis=0)
        sh *= 2
    c_excl = inc - a
    counts = inc[M - 1:M, :]
    off_inc = counts
    sh = 1
    while sh < E:
        off_inc = off_inc + jnp.concatenate(
            [jnp.zeros((1, sh), jnp.float32), off_inc[:, :E - sh]], axis=1)
        sh *= 2
    offsets = off_inc - counts

    dmat = c_excl + offsets
    d1_ref[...] = jnp.sum(nh1 * dmat, axis=1).astype(jnp.int32)
    d2_ref[...] = jnp.sum(nh2 * dmat, axis=1).astype(jnp.int32)

    b_lo = lax.broadcasted_iota(jnp.int32, (NB, 1), 0).astype(
        jnp.float32) * BM
    ends = offsets + counts
    present = ((offsets < b_lo + BM) & (ends > b_lo)
               & (counts > 0)).astype(jnp.float32)
    pe = present
    sh = 1
    while sh < E:
        pe = pe + jnp.concatenate(
            [jnp.zeros((NB, sh), jnp.float32), pe[:, :E - sh]], axis=1)
        sh *= 2
    nb_tiles = jnp.reshape(pe[:, E - 1], (1, NB))
    st = nb_tiles
    sh = 1
    while sh < NB:
        st = st + jnp.concatenate(
            [jnp.zeros((1, sh), jnp.float32), st[:, :NB - sh]], axis=1)
        sh *= 2
    st = st - nb_tiles
    total = st[:, NB - 1:NB] + nb_tiles[:, NB - 1:NB]

    t_col = lax.broadcasted_iota(jnp.int32, (NT, 1), 0).astype(
        jnp.float32)
    ge = (st <= t_col).astype(jnp.float32)
    bt = jnp.sum(ge, axis=1, keepdims=True) - 1.0
    bt = jnp.clip(bt, 0.0, NB - 1.0)
    oh_b = (lax.broadcasted_iota(jnp.int32, (NT, NB), 1).astype(jnp.float32)
            == bt).astype(jnp.float32)
    st_bt = jnp.sum(oh_b * st, axis=1, keepdims=True)
    r_t = t_col - st_bt
    pebt = lax.dot_general(oh_b, pe, (((1,), (0,)), ((), ())),
                           preferred_element_type=jnp.float32)
    et = jnp.sum((pebt <= r_t).astype(jnp.float32), axis=1,
                 keepdims=True)
    et = jnp.clip(et, 0.0, E - 1.0)
    valid = (t_col < total).astype(jnp.float32)
    bt_prev = jnp.concatenate([-jnp.ones((1, 1), jnp.float32),
                               bt[:NT - 1]], axis=0)
    first = (bt != bt_prev).astype(jnp.float32)
    tmeta_ref[0, :] = jnp.reshape(bt, (NT,)).astype(jnp.int32)
    tmeta_ref[1, :] = jnp.reshape(et, (NT,)).astype(jnp.int32)
    tmeta_ref[2, :] = jnp.reshape(valid, (NT,)).astype(jnp.int32)
    tmeta_ref[3, :] = jnp.reshape(first, (NT,)).astype(jnp.int32)

    zpad = jnp.zeros((M, CWL - E), dtype=jnp.float32)
    cw1_ref[...] = jnp.concatenate([nh1 * t1, zpad], axis=1)
    cw2_ref[...] = jnp.concatenate([nh2 * t2, zpad], axis=1)


def _router_call(x, gate_w):
    return pl.pallas_call(
        _router_body,
        grid=(1,),
        in_specs=[
            pl.BlockSpec((M, H), lambda i: (0, 0)),
            pl.BlockSpec((E, H), lambda i: (0, 0)),
        ],
        out_specs=[
            pl.BlockSpec((M,), lambda i: (0,)),
            pl.BlockSpec((M,), lambda i: (0,)),
            pl.BlockSpec((M, CWL), lambda i: (0, 0)),
            pl.BlockSpec((M, CWL), lambda i: (0, 0)),
            pl.BlockSpec((4, NT), lambda i: (0, 0)),
        ],
        out_shape=[
            jax.ShapeDtypeStruct((M,), jnp.int32),
            jax.ShapeDtypeStruct((M,), jnp.int32),
            jax.ShapeDtypeStruct((M, CWL), jnp.float32),
            jax.ShapeDtypeStruct((M, CWL), jnp.float32),
            jax.ShapeDtypeStruct((4, NT), jnp.int32),
        ],
        compiler_params=pltpu.CompilerParams(
            vmem_limit_bytes=100 * 1024 * 1024,
        ),
    )(x, gate_w)


@functools.lru_cache(maxsize=None)
def _sc_mesh():
    return plsc.VectorSubcoreMesh(core_axis_name="c", subcore_axis_name="s",
                                  num_cores=NC, num_subcores=NS)


@functools.lru_cache(maxsize=None)
def _dispatch_fn():
    @functools.partial(
        pl.kernel,
        out_type=[
            jax.ShapeDtypeStruct((P, H), jnp.float32),
            jax.ShapeDtypeStruct((P, CWL), jnp.float32),
        ],
        mesh=_sc_mesh(),
        scratch_types=[
            pltpu.VMEM((TPW, H), jnp.float32),
            pltpu.VMEM((TPW, CWL), jnp.float32),
            pltpu.VMEM((TPW,), jnp.int32),
            pltpu.VMEM((TPW,), jnp.int32),
            pltpu.SemaphoreType.DMA,
            pltpu.SemaphoreType.DMA,
        ],
    )
    def _dispatch(x_hbm, d1_hbm, d2_hbm, cw1_hbm, cw2_hbm, xs_hbm, cws_hbm,
                  xrows, cwbuf, idx1, idx2, sem1, sem2):
        wid = lax.axis_index("s") * NC + lax.axis_index("c")
        base = wid * TPW
        pltpu.sync_copy(d1_hbm.at[pl.ds(base, TPW)], idx1)
        pltpu.sync_copy(d2_hbm.at[pl.ds(base, TPW)], idx2)
        pltpu.sync_copy(x_hbm.at[pl.ds(base, TPW)], xrows)
        c1 = pltpu.async_copy(xrows, xs_hbm.at[idx1], sem1)
        c2 = pltpu.async_copy(xrows, xs_hbm.at[idx2], sem2)
        pltpu.sync_copy(cw1_hbm.at[pl.ds(base, TPW)], cwbuf)
        c1.wait()
        pltpu.async_copy(cwbuf, cws_hbm.at[idx1], sem1).wait()
        pltpu.sync_copy(cw2_hbm.at[pl.ds(base, TPW)], cwbuf)
        c2.wait()
        pltpu.async_copy(cwbuf, cws_hbm.at[idx2], sem2).wait()

    return _dispatch


def _gmm_body(tm_ref, xs_ref, cw_ref, w13_ref, w2_ref, ys_ref):
    t = pl.program_id(0)
    e = tm_ref[1, t]
    valid = tm_ref[2, t]
    first = tm_ref[3, t]

    @pl.when(first == 1)
    def _():
        ys_ref[...] = jnp.zeros((BM, H), jnp.float32)

    @pl.when(valid == 1)
    def _():
        xb = xs_ref[...].astype(jnp.bfloat16)
        li = lax.broadcasted_iota(jnp.int32, (BM, E), 1)
        sc = jnp.sum(cw_ref[:, :E] * (li == e), axis=1)
        gu = lax.dot_general(xb, w13_ref[0], (((1,), (1,)), ((), ())),
                             preferred_element_type=jnp.float32)
        g = gu[:, :I]
        u = gu[:, I:]
        h = (g * (1.0 / (1.0 + jnp.exp(-g)))) * u * sc[:, None]
        ys_ref[...] += lax.dot_general(
            h.astype(jnp.bfloat16), w2_ref[0], (((1,), (1,)), ((), ())),
            preferred_element_type=jnp.float32)


def _gmm_call(tmeta, xs, cws, w13b, w2b):
    grid_spec = pltpu.PrefetchScalarGridSpec(
        num_scalar_prefetch=1,
        grid=(NT,),
        in_specs=[
            pl.BlockSpec((BM, H), lambda t, m: (m[0, t], 0)),
            pl.BlockSpec((BM, CWL), lambda t, m: (m[0, t], 0)),
            pl.BlockSpec((1, 2 * I, H), lambda t, m: (m[1, t], 0, 0)),
            pl.BlockSpec((1, H, I), lambda t, m: (m[1, t], 0, 0)),
        ],
        out_specs=pl.BlockSpec((BM, H), lambda t, m: (m[0, t], 0)),
    )
    return pl.pallas_call(
        _gmm_body,
        grid_spec=grid_spec,
        out_shape=jax.ShapeDtypeStruct((P, H), jnp.float32),
        compiler_params=pltpu.CompilerParams(
            vmem_limit_bytes=100 * 1024 * 1024,
        ),
    )(tmeta, xs, cws, w13b, w2b)


TPW2 = TPW // 2


@functools.lru_cache(maxsize=None)
def _combine_fn():
    @functools.partial(
        pl.kernel,
        out_type=jax.ShapeDtypeStruct((M, H), jnp.float32),
        mesh=_sc_mesh(),
        scratch_types=[
            pltpu.VMEM((TPW2, H), jnp.float32),
            pltpu.VMEM((TPW2, H), jnp.float32),
            pltpu.VMEM((TPW,), jnp.int32),
            pltpu.VMEM((TPW,), jnp.int32),
            pltpu.SemaphoreType.DMA,
            pltpu.SemaphoreType.DMA,
        ],
    )
    def _combine(ys_hbm, d1_hbm, d2_hbm, out_hbm, acc, buf2, idx1, idx2,
                 sem1, sem2):
        wid = lax.axis_index("s") * NC + lax.axis_index("c")
        base = wid * TPW
        pltpu.sync_copy(d1_hbm.at[pl.ds(base, TPW)], idx1)
        pltpu.sync_copy(d2_hbm.at[pl.ds(base, TPW)], idx2)
        for half in range(2):
            ia = idx1.at[pl.ds(half * TPW2, TPW2)]
            ib = idx2.at[pl.ds(half * TPW2, TPW2)]
            c1 = pltpu.async_copy(ys_hbm.at[ia], acc, sem1)
            c2 = pltpu.async_copy(ys_hbm.at[ib], buf2, sem2)
            c1.wait()
            c2.wait()

            def body(c, _):
                col = c * 16
                for r in range(TPW2):
                    acc[r, pl.ds(col, 16)] = (acc[r, pl.ds(col, 16)]
                                              + buf2[r, pl.ds(col, 16)])
                return 0

            lax.fori_loop(0, H // 16, body, 0)
            pltpu.sync_copy(acc,
                            out_hbm.at[pl.ds(base + half * TPW2, TPW2)])

    return _combine


@jax.jit
def kernel(hidden_states, gate_w, w13, w2):
    x = hidden_states.reshape(-1, H)
    w13b = w13.astype(jnp.bfloat16)
    w2b = w2.astype(jnp.bfloat16)
    d1, d2, cw1, cw2, tmeta = _router_call(x, gate_w)
    xs, cws = _dispatch_fn()(x, d1, d2, cw1, cw2)
    ys = _gmm_call(tmeta, xs, cws, w13b, w2b)
    out = _combine_fn()(ys, d1, d2)
    return out.reshape(hidden_states.shape)


# speedup vs baseline: 2.1145x; 1.0569x over previous
import functools

import jax
import jax.numpy as jnp
from jax import lax
from jax.experimental import pallas as pl
from jax.experimental.pallas import tpu as pltpu
from jax.experimental.pallas import tpu_sc as plsc

M = 2048
H = 1024
I = 768
E = 8
K = 2
P = M * K
CH = 256
NCH = M // CH
BM = 256
CWL = 128
NB = P // BM
NT = NB + E - 1

NC, NS = 2, 16
NW = NC * NS
TPW = M // NW


def _router_body(x_ref, gate_ref, d1_ref, d2_ref, cw1_ref, cw2_ref,
                 tmeta_ref, xp_ref):
    x = x_ref[...]
    gate_w = gate_ref[...]
    logits = lax.dot_general(x, gate_w, (((1,), (1,)), ((), ())),
                             preferred_element_type=jnp.float32)
    idxE = lax.broadcasted_iota(jnp.int32, (M, E), 1)
    m1 = jnp.max(logits, axis=1, keepdims=True)
    id1 = jnp.min(jnp.where(logits == m1, idxE, E), axis=1, keepdims=True)
    masked = jnp.where(idxE == id1, -jnp.inf, logits)
    m2 = jnp.max(masked, axis=1, keepdims=True)
    id2 = jnp.min(jnp.where(masked == m2, idxE, E), axis=1, keepdims=True)
    r = jnp.exp(m2 - m1)
    t1 = 1.0 / (1.0 + r)
    t2 = r / (1.0 + r)

    nh1 = (idxE == id1).astype(jnp.float32)
    nh2 = (idxE == id2).astype(jnp.float32)
    a = nh1 + nh2

    inc = a
    sh = 1
    while sh < M:
        inc = inc + jnp.concatenate(
            [jnp.zeros((sh, E), jnp.float32), inc[:M - sh]], axis=0)
        sh *= 2
    c_excl = inc - a
    counts = inc[M - 1:M, :]
    off_inc = counts
    sh = 1
    while sh < E:
        off_inc = off_inc + jnp.concatenate(
            [jnp.zeros((1, sh), jnp.float32), off_inc[:, :E - sh]], axis=1)
        sh *= 2
    offsets = off_inc - counts

    dmat = c_excl + offsets
    d1_ref[...] = jnp.sum(nh1 * dmat, axis=1).astype(jnp.int32)
    d2_ref[...] = jnp.sum(nh2 * dmat, axis=1).astype(jnp.int32)

    b_lo = lax.broadcasted_iota(jnp.int32, (NB, 1), 0).astype(
        jnp.float32) * BM
    ends = offsets + counts
    present = ((offsets < b_lo + BM) & (ends > b_lo)
               & (counts > 0)).astype(jnp.float32)
    pe = present
    sh = 1
    while sh < E:
        pe = pe + jnp.concatenate(
            [jnp.zeros((NB, sh), jnp.float32), pe[:, :E - sh]], axis=1)
        sh *= 2
    nb_tiles = jnp.reshape(pe[:, E - 1], (1, NB))
    st = nb_tiles
    sh = 1
    while sh < NB:
        st = st + jnp.concatenate(
            [jnp.zeros((1, sh), jnp.float32), st[:, :NB - sh]], axis=1)
        sh *= 2
    st = st - nb_tiles
    total = st[:, NB - 1:NB] + nb_tiles[:, NB - 1:NB]

    t_col = lax.broadcasted_iota(jnp.int32, (NT, 1), 0).astype(
        jnp.float32)
    ge = (st <= t_col).astype(jnp.float32)
    bt = jnp.sum(ge, axis=1, keepdims=True) - 1.0
    bt = jnp.clip(bt, 0.0, NB - 1.0)
    oh_b = (lax.broadcasted_iota(jnp.int32, (NT, NB), 1).astype(jnp.float32)
            == bt).astype(jnp.float32)
    st_bt = jnp.sum(oh_b * st, axis=1, keepdims=True)
    r_t = t_col - st_bt
    pebt = lax.dot_general(oh_b, pe, (((1,), (0,)), ((), ())),
                           preferred_element_type=jnp.float32)
    et = jnp.sum((pebt <= r_t).astype(jnp.float32), axis=1,
                 keepdims=True)
    et = jnp.clip(et, 0.0, E - 1.0)
    valid = (t_col < total).astype(jnp.float32)
    bt_prev = jnp.concatenate([-jnp.ones((1, 1), jnp.float32),
                               bt[:NT - 1]], axis=0)
    first = (bt != bt_prev).astype(jnp.float32)
    tmeta_ref[0, :] = jnp.reshape(bt, (NT,)).astype(jnp.int32)
    tmeta_ref[1, :] = jnp.reshape(et, (NT,)).astype(jnp.int32)
    tmeta_ref[2, :] = jnp.reshape(valid, (NT,)).astype(jnp.int32)
    tmeta_ref[3, :] = jnp.reshape(first, (NT,)).astype(jnp.int32)

    zpad = jnp.zeros((M, CWL - E), dtype=jnp.float32)
    cw1_ref[...] = jnp.concatenate([nh1 * t1, zpad], axis=1)
    cw2_ref[...] = jnp.concatenate([nh2 * t2, zpad], axis=1)

    xbf = x.astype(jnp.bfloat16)
    lo = lax.bitcast_convert_type(xbf[:, :H // 2], jnp.uint16).astype(
        jnp.int32)
    hi = lax.bitcast_convert_type(xbf[:, H // 2:], jnp.uint16).astype(
        jnp.int32)
    xp_ref[...] = lax.shift_left(hi, 16) | lo


def _router_call(x, gate_w):
    return pl.pallas_call(
        _router_body,
        grid=(1,),
        in_specs=[
            pl.BlockSpec((M, H), lambda i: (0, 0)),
            pl.BlockSpec((E, H), lambda i: (0, 0)),
        ],
        out_specs=[
            pl.BlockSpec((M,), lambda i: (0,)),
            pl.BlockSpec((M,), lambda i: (0,)),
            pl.BlockSpec((M, CWL), lambda i: (0, 0)),
            pl.BlockSpec((M, CWL), lambda i: (0, 0)),
            pl.BlockSpec((4, NT), lambda i: (0, 0)),
            pl.BlockSpec((M, H // 2), lambda i: (0, 0)),
        ],
        out_shape=[
            jax.ShapeDtypeStruct((M,), jnp.int32),
            jax.ShapeDtypeStruct((M,), jnp.int32),
            jax.ShapeDtypeStruct((M, CWL), jnp.float32),
            jax.ShapeDtypeStruct((M, CWL), jnp.float32),
            jax.ShapeDtypeStruct((4, NT), jnp.int32),
            jax.ShapeDtypeStruct((M, H // 2), jnp.int32),
        ],
        compiler_params=pltpu.CompilerParams(
            vmem_limit_bytes=100 * 1024 * 1024,
        ),
    )(x, gate_w)


@functools.lru_cache(maxsize=None)
def _sc_mesh():
    return plsc.VectorSubcoreMesh(core_axis_name="c", subcore_axis_name="s",
                                  num_cores=NC, num_subcores=NS)


@functools.lru_cache(maxsize=None)
def _dispatch_fn():
    @functools.partial(
        pl.kernel,
        out_type=[
            jax.ShapeDtypeStruct((P, H // 2), jnp.int32),
            jax.ShapeDtypeStruct((P, CWL), jnp.float32),
        ],
        mesh=_sc_mesh(),
        scratch_types=[
            pltpu.VMEM((TPW, H // 2), jnp.int32),
            pltpu.VMEM((TPW, CWL), jnp.float32),
            pltpu.VMEM((TPW,), jnp.int32),
            pltpu.VMEM((TPW,), jnp.int32),
            pltpu.SemaphoreType.DMA,
            pltpu.SemaphoreType.DMA,
        ],
    )
    def _dispatch(x_hbm, d1_hbm, d2_hbm, cw1_hbm, cw2_hbm, xs_hbm, cws_hbm,
                  xrows, cwbuf, idx1, idx2, sem1, sem2):
        wid = lax.axis_index("s") * NC + lax.axis_index("c")
        base = wid * TPW
        pltpu.sync_copy(d1_hbm.at[pl.ds(base, TPW)], idx1)
        pltpu.sync_copy(d2_hbm.at[pl.ds(base, TPW)], idx2)
        pltpu.sync_copy(x_hbm.at[pl.ds(base, TPW)], xrows)
        c1 = pltpu.async_copy(xrows, xs_hbm.at[idx1], sem1)
        c2 = pltpu.async_copy(xrows, xs_hbm.at[idx2], sem2)
        pltpu.sync_copy(cw1_hbm.at[pl.ds(base, TPW)], cwbuf)
        c1.wait()
        pltpu.async_copy(cwbuf, cws_hbm.at[idx1], sem1).wait()
        pltpu.sync_copy(cw2_hbm.at[pl.ds(base, TPW)], cwbuf)
        c2.wait()
        pltpu.async_copy(cwbuf, cws_hbm.at[idx2], sem2).wait()

    return _dispatch


def _gmm_body(tm_ref, xs_ref, cw_ref, w13_ref, w2_ref, ys_ref):
    t = pl.program_id(0)
    e = tm_ref[1, t]
    valid = tm_ref[2, t]
    first = tm_ref[3, t]

    @pl.when(first == 1)
    def _():
        ys_ref[...] = jnp.zeros((BM, H), jnp.float32)

    @pl.when(valid == 1)
    def _():
        xi = xs_ref[...]
        lo = lax.bitcast_convert_type(
            (xi & 0xFFFF).astype(jnp.uint16), jnp.bfloat16)
        hi = lax.bitcast_convert_type(
            lax.shift_right_logical(xi, 16).astype(jnp.uint16), jnp.bfloat16)
        xb = jnp.concatenate([lo, hi], axis=1)
        li = lax.broadcasted_iota(jnp.int32, (BM, E), 1)
        sc = jnp.sum(cw_ref[:, :E] * (li == e), axis=1)
        gu = lax.dot_general(xb, w13_ref[0], (((1,), (1,)), ((), ())),
                             preferred_element_type=jnp.float32)
        g = gu[:, :I]
        u = gu[:, I:]
        h = (g * (1.0 / (1.0 + jnp.exp(-g)))) * u * sc[:, None]
        ys_ref[...] += lax.dot_general(
            h.astype(jnp.bfloat16), w2_ref[0], (((1,), (1,)), ((), ())),
            preferred_element_type=jnp.float32)


def _gmm_call(tmeta, xs, cws, w13b, w2b):
    grid_spec = pltpu.PrefetchScalarGridSpec(
        num_scalar_prefetch=1,
        grid=(NT,),
        in_specs=[
            pl.BlockSpec((BM, H // 2), lambda t, m: (m[0, t], 0)),
            pl.BlockSpec((BM, CWL), lambda t, m: (m[0, t], 0)),
            pl.BlockSpec((1, 2 * I, H), lambda t, m: (m[1, t], 0, 0)),
            pl.BlockSpec((1, H, I), lambda t, m: (m[1, t], 0, 0)),
        ],
        out_specs=pl.BlockSpec((BM, H), lambda t, m: (m[0, t], 0)),
    )
    return pl.pallas_call(
        _gmm_body,
        grid_spec=grid_spec,
        out_shape=jax.ShapeDtypeStruct((P, H), jnp.float32),
        compiler_params=pltpu.CompilerParams(
            vmem_limit_bytes=100 * 1024 * 1024,
        ),
    )(tmeta, xs, cws, w13b, w2b)


NQ = 4
TPQ = TPW // NQ


@functools.lru_cache(maxsize=None)
def _combine_fn():
    @functools.partial(
        pl.kernel,
        out_type=jax.ShapeDtypeStruct((M, H), jnp.float32),
        mesh=_sc_mesh(),
        scratch_types=[
            pltpu.VMEM((TPQ, H), jnp.float32),
            pltpu.VMEM((TPQ, H), jnp.float32),
            pltpu.VMEM((TPQ, H), jnp.float32),
            pltpu.VMEM((TPQ, H), jnp.float32),
            pltpu.VMEM((TPW,), jnp.int32),
            pltpu.VMEM((TPW,), jnp.int32),
            pltpu.SemaphoreType.DMA,
            pltpu.SemaphoreType.DMA,
            pltpu.SemaphoreType.DMA,
            pltpu.SemaphoreType.DMA,
            pltpu.SemaphoreType.DMA,
            pltpu.SemaphoreType.DMA,
        ],
    )
    def _combine(ys_hbm, d1_hbm, d2_hbm, out_hbm, acc0, acc1, buf0, buf1,
                 idx1, idx2, ga0, ga1, gb0, gb1, st0, st1):
        wid = lax.axis_index("s") * NC + lax.axis_index("c")
        base = wid * TPW
        pltpu.sync_copy(d1_hbm.at[pl.ds(base, TPW)], idx1)
        pltpu.sync_copy(d2_hbm.at[pl.ds(base, TPW)], idx2)
        accs, bufs = (acc0, acc1), (buf0, buf1)
        gsems, hsems, ssems = (ga0, ga1), (gb0, gb1), (st0, st1)

        def gathers(q):
            p = q & 1
            ia = idx1.at[pl.ds(q * TPQ, TPQ)]
            ib = idx2.at[pl.ds(q * TPQ, TPQ)]
            return (pltpu.async_copy(ys_hbm.at[ia], accs[p], gsems[p]),
                    pltpu.async_copy(ys_hbm.at[ib], bufs[p], hsems[p]))

        pend_g = {0: gathers(0)}
        pend_store = [None, None]
        for q in range(NQ):
            p = q & 1
            if q + 1 < NQ:
                pn = (q + 1) & 1
                if pend_store[pn] is not None:
                    pend_store[pn].wait()
                    pend_store[pn] = None
                pend_g[q + 1] = gathers(q + 1)
            for c in pend_g.pop(q):
                c.wait()

            def body(c, _, p=p):
                col = c * 16
                for r in range(TPQ):
                    accs[p][r, pl.ds(col, 16)] = (
                        accs[p][r, pl.ds(col, 16)]
                        + bufs[p][r, pl.ds(col, 16)])
                return 0

            lax.fori_loop(0, H // 16, body, 0)
            pend_store[p] = pltpu.async_copy(
                accs[p], out_hbm.at[pl.ds(base + q * TPQ, TPQ)], ssems[p])
        for p in range(2):
            if pend_store[p] is not None:
                pend_store[p].wait()

    return _combine


@jax.jit
def kernel(hidden_states, gate_w, w13, w2):
    x = hidden_states.reshape(-1, H)
    w13b = w13.astype(jnp.bfloat16)
    w2b = w2.astype(jnp.bfloat16)
    d1, d2, cw1, cw2, tmeta, xp = _router_call(x, gate_w)
    xs, cws = _dispatch_fn()(xp, d1, d2, cw1, cw2)
    ys = _gmm_call(tmeta, xs, cws, w13b, w2b)
    out = _combine_fn()(ys, d1, d2)
    return out.reshape(hidden_states.shape)
